# Initial kernel scaffold; baseline (speedup 1.0000x reference)
import jax
import jax.numpy as jnp
from jax.experimental import pallas as pl


def kernel(x_cat, x_num, x_biography, tables, W0, b0, g0, bt0, W1, b1, g1, bt1, W2, b2, g2, bt2, W3, b3, g3, bt3, W_out, b_out):
    raise NotImplementedError("write your pallas kernel here")


# speedup vs baseline: 11.3964x; 11.3964x over previous
import functools

import jax
import jax.numpy as jnp
from jax import lax
from jax.experimental import pallas as pl
from jax.experimental.pallas import tpu as pltpu
from jax.experimental.pallas import tpu_sc as plsc

_B = 16384
_NCAT = 26
_VOCAB = 100
_EDIM = 50
_EPAD = 64
_NNUM = 13
_EPS = 1e-5
_CH = 128


def _make_sc_gather():
    info = plsc.get_sparse_core_info()
    nc, ns = info.num_cores, info.num_subcores
    nw = nc * ns
    rows = _B * _NCAT
    per_w = rows // nw
    nch = per_w // _CH

    mesh = plsc.VectorSubcoreMesh(core_axis_name="c", subcore_axis_name="s")

    @functools.partial(
        pl.kernel,
        mesh=mesh,
        compiler_params=pltpu.CompilerParams(use_tc_tiling_on_sc=False),
        out_type=jax.ShapeDtypeStruct((rows, _EPAD), jnp.float32),
        scratch_types=[
            pltpu.VMEM((nch, _CH), jnp.int32),
            pltpu.VMEM((_CH, _EPAD), jnp.float32),
            pltpu.SemaphoreType.DMA,
        ],
    )
    def gather_k(tab_hbm, idx_hbm, out_hbm, idx_v, buf, sem):
        wid = lax.axis_index("s") * nc + lax.axis_index("c")
        row0 = wid * per_w
        pltpu.sync_copy(idx_hbm.at[wid], idx_v)

        def body(c, carry):
            pltpu.async_copy(tab_hbm.at[idx_v.at[c]], buf, sem).wait()
            pltpu.sync_copy(buf, out_hbm.at[pl.ds(row0 + c * _CH, _CH)])
            return carry

        lax.fori_loop(0, nch, body, 0)

    return gather_k, nw, nch


def _stats_update(i, nb, z, so_ref):
    p = jnp.concatenate(
        [jnp.sum(z, 0, keepdims=True), jnp.sum(z * z, 0, keepdims=True)], 0)

    @pl.when(i == 0)
    def _():
        so_ref[...] = p

    @pl.when(i > 0)
    def _():
        so_ref[...] += p


def _bn_relu(z, s_ref, g_ref, bt_ref, n_rows):
    s = s_ref[...]
    m = s[0] * (1.0 / n_rows)
    var = s[1] * (1.0 / n_rows) - m * m
    alpha = g_ref[...] * lax.rsqrt(var + _EPS)
    beta = bt_ref[...] - m * alpha
    return jnp.maximum(z * alpha + beta, 0.0)


def _pass_first(emb, xn, we, wn, b, bm):
    nb, k = _B // bm, emb.shape[1]
    n = we.shape[1]

    def body(e_ref, xn_ref, we_ref, wn_ref, b_ref, zo_ref, so_ref):
        i = pl.program_id(0)
        z = (jnp.dot(e_ref[...], we_ref[...], preferred_element_type=jnp.float32)
             + jnp.dot(xn_ref[...], wn_ref[...], preferred_element_type=jnp.float32)
             + b_ref[...])
        zo_ref[...] = z
        _stats_update(i, nb, z, so_ref)

    return pl.pallas_call(
        body,
        grid=(nb,),
        in_specs=[
            pl.BlockSpec((bm, k), lambda i: (i, 0)),
            pl.BlockSpec((bm, _NNUM), lambda i: (i, 0)),
            pl.BlockSpec((k, n), lambda i: (0, 0)),
            pl.BlockSpec((_NNUM, n), lambda i: (0, 0)),
            pl.BlockSpec((n,), lambda i: (0,)),
        ],
        out_specs=[
            pl.BlockSpec((bm, n), lambda i: (i, 0)),
            pl.BlockSpec((2, n), lambda i: (0, 0)),
        ],
        out_shape=[
            jax.ShapeDtypeStruct((_B, n), jnp.float32),
            jax.ShapeDtypeStruct((2, n), jnp.float32),
        ],
    )(emb, xn, we, wn, b)


def _pass_mid(z, s, g, bt, w, b, bm):
    nb, k = _B // bm, z.shape[1]
    n = w.shape[1]

    def body(z_ref, s_ref, g_ref, bt_ref, w_ref, b_ref, zo_ref, so_ref):
        i = pl.program_id(0)
        a = _bn_relu(z_ref[...], s_ref, g_ref, bt_ref, _B)
        zz = jnp.dot(a, w_ref[...], preferred_element_type=jnp.float32) + b_ref[...]
        zo_ref[...] = zz
        _stats_update(i, nb, zz, so_ref)

    return pl.pallas_call(
        body,
        grid=(nb,),
        in_specs=[
            pl.BlockSpec((bm, k), lambda i: (i, 0)),
            pl.BlockSpec((2, k), lambda i: (0, 0)),
            pl.BlockSpec((k,), lambda i: (0,)),
            pl.BlockSpec((k,), lambda i: (0,)),
            pl.BlockSpec((k, n), lambda i: (0, 0)),
            pl.BlockSpec((n,), lambda i: (0,)),
        ],
        out_specs=[
            pl.BlockSpec((bm, n), lambda i: (i, 0)),
            pl.BlockSpec((2, n), lambda i: (0, 0)),
        ],
        out_shape=[
            jax.ShapeDtypeStruct((_B, n), jnp.float32),
            jax.ShapeDtypeStruct((2, n), jnp.float32),
        ],
    )(z, s, g, bt, w, b)


def _pass_head(z, s, g, bt, w, b, bm):
    nb, k = _B // bm, z.shape[1]

    def body(z_ref, s_ref, g_ref, bt_ref, w_ref, b_ref, o_ref):
        a = _bn_relu(z_ref[...], s_ref, g_ref, bt_ref, _B)
        o_ref[...] = (jnp.dot(a, w_ref[...], preferred_element_type=jnp.float32)
                      + b_ref[...])

    return pl.pallas_call(
        body,
        grid=(nb,),
        in_specs=[
            pl.BlockSpec((bm, k), lambda i: (i, 0)),
            pl.BlockSpec((2, k), lambda i: (0, 0)),
            pl.BlockSpec((k,), lambda i: (0,)),
            pl.BlockSpec((k,), lambda i: (0,)),
            pl.BlockSpec((k, 1), lambda i: (0, 0)),
            pl.BlockSpec((1,), lambda i: (0,)),
        ],
        out_specs=pl.BlockSpec((bm, 1), lambda i: (i, 0)),
        out_shape=jax.ShapeDtypeStruct((_B, 1), jnp.float32),
    )(z, s, g, bt, w, b)


def kernel(x_cat, x_num, x_biography, tables,
           W0, b0, g0, bt0,
           W1, b1, g1, bt1,
           W2, b2, g2, bt2,
           W3, b3, g3, bt3,
           W_out, b_out):
    del x_biography
    gather_k, nw, nch = _make_sc_gather()

    offs = (jnp.arange(_NCAT, dtype=jnp.int32) * _VOCAB)[None, :]
    idx = (x_cat.astype(jnp.int32) + offs).reshape(nw, nch, _CH)

    tabp = jnp.pad(tables, ((0, 0), (0, 0), (0, _EPAD - _EDIM)))
    tabp = tabp.reshape(_NCAT * _VOCAB, _EPAD)

    emb = gather_k(tabp, idx).reshape(_B, _NCAT * _EPAD)

    w0e = jnp.pad(W0[: _NCAT * _EDIM].reshape(_NCAT, _EDIM, -1),
                  ((0, 0), (0, _EPAD - _EDIM), (0, 0)))
    w0e = w0e.reshape(_NCAT * _EPAD, -1)
    w0n = W0[_NCAT * _EDIM:]

    z0, s0 = _pass_first(emb, x_num, w0e, w0n, b0, bm=256)
    z1, s1 = _pass_mid(z0, s0, g0, bt0, W1, b1, bm=256)
    z2, s2 = _pass_mid(z1, s1, g1, bt1, W2, b2, bm=256)
    z3, s3 = _pass_mid(z2, s2, g2, bt2, W3, b3, bm=256)
    return _pass_head(z3, s3, g3, bt3, W_out, b_out, bm=512)


# speedup vs baseline: 11.4202x; 1.0021x over previous
import functools

import jax
import jax.numpy as jnp
from jax import lax
from jax.experimental import pallas as pl
from jax.experimental.pallas import tpu as pltpu
from jax.experimental.pallas import tpu_sc as plsc

_B = 16384
_NCAT = 26
_VOCAB = 100
_EDIM = 50
_EPAD = 64
_NNUM = 13
_EPS = 1e-5
_CH = 128


def _make_sc_gather():
    info = plsc.get_sparse_core_info()
    nc, ns = info.num_cores, info.num_subcores
    nw = nc * ns
    rows = _B * _NCAT
    per_w = rows // nw
    nch = per_w // _CH

    mesh = plsc.VectorSubcoreMesh(core_axis_name="c", subcore_axis_name="s")

    @functools.partial(
        pl.kernel,
        mesh=mesh,
        compiler_params=pltpu.CompilerParams(use_tc_tiling_on_sc=False),
        out_type=jax.ShapeDtypeStruct((rows, _EPAD), jnp.float32),
        scratch_types=[
            pltpu.VMEM((nch, _CH), jnp.int32),
            pltpu.VMEM((_CH, _EPAD), jnp.float32),
            pltpu.SemaphoreType.DMA,
        ],
    )
    def gather_k(tab_hbm, idx_hbm, out_hbm, idx_v, buf, sem):
        wid = lax.axis_index("s") * nc + lax.axis_index("c")
        row0 = wid * per_w
        pltpu.sync_copy(idx_hbm.at[wid], idx_v)

        def body(c, carry):
            pltpu.async_copy(tab_hbm.at[idx_v.at[c]], buf, sem).wait()
            pltpu.sync_copy(buf, out_hbm.at[pl.ds(row0 + c * _CH, _CH)])
            return carry

        lax.fori_loop(0, nch, body, 0)

    return gather_k, nw, nch


def _dot16(a, w):
    return jnp.dot(a.astype(jnp.bfloat16), w.astype(jnp.bfloat16),
                   preferred_element_type=jnp.float32)


def _stats_update(i, nb, z, so_ref):
    p = jnp.concatenate(
        [jnp.sum(z, 0, keepdims=True), jnp.sum(z * z, 0, keepdims=True)], 0)

    @pl.when(i == 0)
    def _():
        so_ref[...] = p

    @pl.when(i > 0)
    def _():
        so_ref[...] += p


def _bn_relu(z, s_ref, g_ref, bt_ref, n_rows):
    s = s_ref[...]
    m = s[0] * (1.0 / n_rows)
    var = s[1] * (1.0 / n_rows) - m * m
    alpha = g_ref[...] * lax.rsqrt(var + _EPS)
    beta = bt_ref[...] - m * alpha
    return jnp.maximum(z * alpha + beta, 0.0)


def _pass_first(emb, xn, we, wn, b, bm):
    nb, k = _B // bm, emb.shape[1]
    n = we.shape[1]

    def body(e_ref, xn_ref, we_ref, wn_ref, b_ref, zo_ref, so_ref):
        i = pl.program_id(0)
        z = (_dot16(e_ref[...], we_ref[...])
             + _dot16(xn_ref[...], wn_ref[...])
             + b_ref[...])
        zo_ref[...] = z
        _stats_update(i, nb, z, so_ref)

    return pl.pallas_call(
        body,
        grid=(nb,),
        in_specs=[
            pl.BlockSpec((bm, k), lambda i: (i, 0)),
            pl.BlockSpec((bm, _NNUM), lambda i: (i, 0)),
            pl.BlockSpec((k, n), lambda i: (0, 0)),
            pl.BlockSpec((_NNUM, n), lambda i: (0, 0)),
            pl.BlockSpec((n,), lambda i: (0,)),
        ],
        out_specs=[
            pl.BlockSpec((bm, n), lambda i: (i, 0)),
            pl.BlockSpec((2, n), lambda i: (0, 0)),
        ],
        out_shape=[
            jax.ShapeDtypeStruct((_B, n), jnp.float32),
            jax.ShapeDtypeStruct((2, n), jnp.float32),
        ],
    )(emb, xn, we, wn, b)


def _pass_mid(z, s, g, bt, w, b, bm):
    nb, k = _B // bm, z.shape[1]
    n = w.shape[1]

    def body(z_ref, s_ref, g_ref, bt_ref, w_ref, b_ref, zo_ref, so_ref):
        i = pl.program_id(0)
        a = _bn_relu(z_ref[...], s_ref, g_ref, bt_ref, _B)
        zz = _dot16(a, w_ref[...]) + b_ref[...]
        zo_ref[...] = zz
        _stats_update(i, nb, zz, so_ref)

    return pl.pallas_call(
        body,
        grid=(nb,),
        in_specs=[
            pl.BlockSpec((bm, k), lambda i: (i, 0)),
            pl.BlockSpec((2, k), lambda i: (0, 0)),
            pl.BlockSpec((k,), lambda i: (0,)),
            pl.BlockSpec((k,), lambda i: (0,)),
            pl.BlockSpec((k, n), lambda i: (0, 0)),
            pl.BlockSpec((n,), lambda i: (0,)),
        ],
        out_specs=[
            pl.BlockSpec((bm, n), lambda i: (i, 0)),
            pl.BlockSpec((2, n), lambda i: (0, 0)),
        ],
        out_shape=[
            jax.ShapeDtypeStruct((_B, n), jnp.float32),
            jax.ShapeDtypeStruct((2, n), jnp.float32),
        ],
    )(z, s, g, bt, w, b)


def _pass_head(z, s, g, bt, w, b, bm):
    nb, k = _B // bm, z.shape[1]

    def body(z_ref, s_ref, g_ref, bt_ref, w_ref, b_ref, o_ref):
        a = _bn_relu(z_ref[...], s_ref, g_ref, bt_ref, _B)
        o_ref[...] = _dot16(a, w_ref[...]) + b_ref[...]

    return pl.pallas_call(
        body,
        grid=(nb,),
        in_specs=[
            pl.BlockSpec((bm, k), lambda i: (i, 0)),
            pl.BlockSpec((2, k), lambda i: (0, 0)),
            pl.BlockSpec((k,), lambda i: (0,)),
            pl.BlockSpec((k,), lambda i: (0,)),
            pl.BlockSpec((k, 1), lambda i: (0, 0)),
            pl.BlockSpec((1,), lambda i: (0,)),
        ],
        out_specs=pl.BlockSpec((bm, 1), lambda i: (i, 0)),
        out_shape=jax.ShapeDtypeStruct((_B, 1), jnp.float32),
    )(z, s, g, bt, w, b)


def kernel(x_cat, x_num, x_biography, tables,
           W0, b0, g0, bt0,
           W1, b1, g1, bt1,
           W2, b2, g2, bt2,
           W3, b3, g3, bt3,
           W_out, b_out):
    del x_biography
    gather_k, nw, nch = _make_sc_gather()

    offs = (jnp.arange(_NCAT, dtype=jnp.int32) * _VOCAB)[None, :]
    idx = (x_cat.astype(jnp.int32) + offs).reshape(nw, nch, _CH)

    tabp = jnp.pad(tables, ((0, 0), (0, 0), (0, _EPAD - _EDIM)))
    tabp = tabp.reshape(_NCAT * _VOCAB, _EPAD)

    emb = gather_k(tabp, idx).reshape(_B, _NCAT * _EPAD)

    w0e = jnp.pad(W0[: _NCAT * _EDIM].reshape(_NCAT, _EDIM, -1),
                  ((0, 0), (0, _EPAD - _EDIM), (0, 0)))
    w0e = w0e.reshape(_NCAT * _EPAD, -1)
    w0n = W0[_NCAT * _EDIM:]

    z0, s0 = _pass_first(emb, x_num, w0e, w0n, b0, bm=256)
    z1, s1 = _pass_mid(z0, s0, g0, bt0, W1, b1, bm=256)
    z2, s2 = _pass_mid(z1, s1, g1, bt1, W2, b2, bm=256)
    z3, s3 = _pass_mid(z2, s2, g2, bt2, W3, b3, bm=256)
    return _pass_head(z3, s3, g3, bt3, W_out, b_out, bm=512)
